# baseline (speedup 1.0000x reference)
import jax
import jax.numpy as jnp
from jax.experimental import pallas as pl
from jax.experimental.pallas import tpu as pltpu

_BLOCK_ROWS = 400
_NBUF = 2


def _dot(a, b):
    return jax.lax.dot_general(
        a, b, (((1,), (0,)), ((), ())), preferred_element_type=jnp.float32
    )


def _gcn_body(adj_hbm, x_ref, w_ref, out_ref, bufs, support_ref, sems):
    n_nodes = adj_hbm.shape[0]
    br = _BLOCK_ROWS
    nblk = n_nodes // br

    def start(i, b):
        pltpu.make_async_copy(
            adj_hbm.at[pl.ds(i * br, br), :], bufs.at[b], sems.at[b]
        ).start()

    def wait(i, b):
        pltpu.make_async_copy(
            adj_hbm.at[pl.ds(i * br, br), :], bufs.at[b], sems.at[b]
        ).wait()

    for b in range(_NBUF):
        start(b, b)

    support_ref[...] = _dot(x_ref[...], w_ref[...])

    def outer(g, carry):
        base = g * _NBUF
        for b in range(_NBUF):
            i = base + b
            wait(i, b)
            out_ref[pl.ds(i * br, br), :] = _dot(bufs[b], support_ref[...])
            nxt = i + _NBUF

            @pl.when(nxt < nblk)
            def _():
                start(nxt, b)
        return carry

    jax.lax.fori_loop(0, nblk // _NBUF, outer, 0)
    for b in range(nblk % _NBUF):
        i = (nblk // _NBUF) * _NBUF + b
        wait(i, b)
        out_ref[pl.ds(i * br, br), :] = _dot(bufs[b], support_ref[...])


def kernel(x, adj, weight):
    n_nodes, f_in = x.shape
    f_out = weight.shape[1]
    return pl.pallas_call(
        _gcn_body,
        in_specs=[
            pl.BlockSpec(memory_space=pl.ANY),
            pl.BlockSpec((n_nodes, f_in), lambda: (0, 0)),
            pl.BlockSpec((f_in, f_out), lambda: (0, 0)),
        ],
        out_specs=pl.BlockSpec((n_nodes, f_out), lambda: (0, 0)),
        out_shape=jax.ShapeDtypeStruct((n_nodes, f_out), jnp.float32),
        scratch_shapes=[
            pltpu.VMEM((_NBUF, _BLOCK_ROWS, n_nodes), jnp.float32),
            pltpu.VMEM((n_nodes, f_out), jnp.float32),
            pltpu.SemaphoreType.DMA((_NBUF,)),
        ],
    )(adj, x, weight)


# speedup vs baseline: 1.0444x; 1.0444x over previous
import jax
import jax.numpy as jnp
from jax.experimental import pallas as pl
from jax.experimental.pallas import tpu as pltpu

_BLOCK_ROWS = 400


def _gcn_body(adj_ref, x_ref, w_ref, out_ref, support_ref):
    @pl.when(pl.program_id(0) == 0)
    def _():
        support_ref[...] = jax.lax.dot_general(
            x_ref[...], w_ref[...],
            (((1,), (0,)), ((), ())),
            preferred_element_type=jnp.float32,
        )

    out_ref[...] = jax.lax.dot_general(
        adj_ref[...], support_ref[...],
        (((1,), (0,)), ((), ())),
        preferred_element_type=jnp.float32,
    )


def kernel(x, adj, weight):
    n_nodes, f_in = x.shape
    f_out = weight.shape[1]
    br = _BLOCK_ROWS
    grid = (n_nodes + br - 1) // br
    return pl.pallas_call(
        _gcn_body,
        grid=(grid,),
        in_specs=[
            pl.BlockSpec((br, n_nodes), lambda i: (i, 0)),
            pl.BlockSpec((n_nodes, f_in), lambda i: (0, 0)),
            pl.BlockSpec((f_in, f_out), lambda i: (0, 0)),
        ],
        out_specs=pl.BlockSpec((br, f_out), lambda i: (i, 0)),
        out_shape=jax.ShapeDtypeStruct((n_nodes, f_out), jnp.float32),
        scratch_shapes=[pltpu.VMEM((n_nodes, f_out), jnp.float32)],
        compiler_params=pltpu.CompilerParams(
            dimension_semantics=("arbitrary",),
        ),
    )(adj, x, weight)
